# Initial kernel scaffold; baseline (speedup 1.0000x reference)
import jax
import jax.numpy as jnp
from jax.experimental import pallas as pl


def kernel(x, router_logits, w1, w1_up, w2):
    raise NotImplementedError("write your pallas kernel here")


# speedup vs baseline: 1.0138x; 1.0138x over previous
import jax
import jax.numpy as jnp
from jax.experimental import pallas as pl
from jax.experimental.pallas import tpu as pltpu

E = 16
TOP_K = 8
D = 2048
F = 1408
B = 32
C = 352
NF = F // C


def _ffn_kernel(rl_ref, xt_ref, w1_ref, w1u_ref, w2_ref, out_ref,
                xt_bf, combt, acct, mixed):
    e = pl.program_id(0)
    f = pl.program_id(1)

    @pl.when((e == 0) & (f == 0))
    def _init():
        xt_bf[:, :] = xt_ref[:, :].astype(jnp.bfloat16)
        acct[:, :] = jnp.zeros((D, B), jnp.float32)
        logits = rl_ref[:, :]
        vals = logits
        sel = jnp.zeros((B, E), jnp.float32)
        idx = jax.lax.broadcasted_iota(jnp.int32, (B, E), 1)
        for _ in range(TOP_K):
            am = jnp.argmax(vals, axis=1)
            first = idx == am[:, None]
            sel = jnp.where(first, 1.0, sel)
            vals = jnp.where(first, -jnp.inf, vals)
        mx = jnp.max(logits, axis=1, keepdims=True)
        ew = jnp.exp(logits - mx) * sel
        w = ew / jnp.sum(ew, axis=1, keepdims=True)
        combt[:, :] = w.T

    w1c = w1_ref[0].astype(jnp.bfloat16)
    w1uc = w1u_ref[0].astype(jnp.bfloat16)
    xtb = xt_bf[:, :]
    gt = jax.lax.dot_general(w1c, xtb, (((1,), (0,)), ((), ())),
                             preferred_element_type=jnp.float32)
    ut = jax.lax.dot_general(w1uc, xtb, (((1,), (0,)), ((), ())),
                             preferred_element_type=jnp.float32)
    cw = combt[pl.ds(e, 1), :]
    mt = gt * jax.lax.logistic(gt) * ut * cw
    mixed[pl.ds(f * C, C), :] = mt.astype(jnp.bfloat16)

    @pl.when(f == NF - 1)
    def _down():
        w2c = w2_ref[0].astype(jnp.bfloat16)
        acct[:, :] += jax.lax.dot_general(
            w2c, mixed[:, :], (((1,), (0,)), ((), ())),
            preferred_element_type=jnp.float32)

    @pl.when((e == E - 1) & (f == NF - 1))
    def _fin():
        out_ref[:, :] = acct[:, :]


def kernel(x, router_logits, w1, w1_up, w2):
    if x.ndim == 2:
        x = x[:, None, :]
    curr = x[:, -1, :]
    outt = pl.pallas_call(
        _ffn_kernel,
        grid=(E, NF),
        in_specs=[
            pl.BlockSpec((B, E), lambda e, f: (0, 0)),
            pl.BlockSpec((D, B), lambda e, f: (0, 0)),
            pl.BlockSpec((1, C, D), lambda e, f: (e, f, 0)),
            pl.BlockSpec((1, C, D), lambda e, f: (e, f, 0)),
            pl.BlockSpec((1, D, F), lambda e, f: (e, 0, 0)),
        ],
        out_specs=pl.BlockSpec((D, B), lambda e, f: (0, 0)),
        out_shape=jax.ShapeDtypeStruct((D, B), jnp.float32),
        scratch_shapes=[
            pltpu.VMEM((D, B), jnp.bfloat16),
            pltpu.VMEM((E, B), jnp.float32),
            pltpu.VMEM((D, B), jnp.float32),
            pltpu.VMEM((F, B), jnp.bfloat16),
        ],
        compiler_params=pltpu.CompilerParams(
            dimension_semantics=("arbitrary", "arbitrary")),
    )(router_logits, curr.T, w1, w1_up, w2)
    return outt.T.reshape(x.shape[0], 1, D)


# speedup vs baseline: 1.0725x; 1.0579x over previous
import jax
import jax.numpy as jnp
from jax.experimental import pallas as pl
from jax.experimental.pallas import tpu as pltpu

E = 16
TOP_K = 8
D = 2048
F = 1408
B = 32
C = 704
NF = F // C


def _ffn_kernel(rl_ref, xt_ref, w1_ref, w1u_ref, w2_ref, out_ref,
                xt_bf, combt, acct, mixed):
    e = pl.program_id(0)
    f = pl.program_id(1)

    @pl.when((e == 0) & (f == 0))
    def _init():
        xt_bf[:, :] = xt_ref[:, :].astype(jnp.bfloat16)
        acct[:, :] = jnp.zeros((D, B), jnp.float32)
        logits = rl_ref[:, :]
        vals = logits
        sel = jnp.zeros((B, E), jnp.float32)
        idx = jax.lax.broadcasted_iota(jnp.int32, (B, E), 1)
        for _ in range(TOP_K):
            am = jnp.argmax(vals, axis=1)
            first = idx == am[:, None]
            sel = jnp.where(first, 1.0, sel)
            vals = jnp.where(first, -jnp.inf, vals)
        mx = jnp.max(logits, axis=1, keepdims=True)
        ew = jnp.exp(logits - mx) * sel
        w = ew / jnp.sum(ew, axis=1, keepdims=True)
        combt[:, :] = w.T

    w1c = w1_ref[0].astype(jnp.bfloat16)
    w1uc = w1u_ref[0].astype(jnp.bfloat16)
    xtb = xt_bf[:, :]
    gt = jax.lax.dot_general(w1c, xtb, (((1,), (0,)), ((), ())),
                             preferred_element_type=jnp.float32)
    ut = jax.lax.dot_general(w1uc, xtb, (((1,), (0,)), ((), ())),
                             preferred_element_type=jnp.float32)
    cw = combt[pl.ds(e, 1), :]
    mt = gt * jax.lax.logistic(gt) * ut * cw
    mixed[pl.ds(f * C, C), :] = mt.astype(jnp.bfloat16)

    @pl.when(f == NF - 1)
    def _down():
        w2c = w2_ref[0].astype(jnp.bfloat16)
        acct[:, :] += jax.lax.dot_general(
            w2c, mixed[:, :], (((1,), (0,)), ((), ())),
            preferred_element_type=jnp.float32)

    @pl.when((e == E - 1) & (f == NF - 1))
    def _fin():
        out_ref[:, :] = acct[:, :]


def kernel(x, router_logits, w1, w1_up, w2):
    if x.ndim == 2:
        x = x[:, None, :]
    curr = x[:, -1, :]
    outt = pl.pallas_call(
        _ffn_kernel,
        grid=(E, NF),
        in_specs=[
            pl.BlockSpec((B, E), lambda e, f: (0, 0)),
            pl.BlockSpec((D, B), lambda e, f: (0, 0)),
            pl.BlockSpec((1, C, D), lambda e, f: (e, f, 0)),
            pl.BlockSpec((1, C, D), lambda e, f: (e, f, 0)),
            pl.BlockSpec((1, D, F), lambda e, f: (e, 0, 0)),
        ],
        out_specs=pl.BlockSpec((D, B), lambda e, f: (0, 0)),
        out_shape=jax.ShapeDtypeStruct((D, B), jnp.float32),
        scratch_shapes=[
            pltpu.VMEM((D, B), jnp.bfloat16),
            pltpu.VMEM((E, B), jnp.float32),
            pltpu.VMEM((D, B), jnp.float32),
            pltpu.VMEM((F, B), jnp.bfloat16),
        ],
        compiler_params=pltpu.CompilerParams(
            dimension_semantics=("arbitrary", "arbitrary")),
    )(router_logits, curr.T, w1, w1_up, w2)
    return outt.T.reshape(x.shape[0], 1, D)
